# Initial kernel scaffold; baseline (speedup 1.0000x reference)
import jax
import jax.numpy as jnp
from jax.experimental import pallas as pl


def kernel(x, weight):
    raise NotImplementedError("write your pallas kernel here")


# speedup vs baseline: 1.1048x; 1.1048x over previous
import functools

import jax
import jax.numpy as jnp
from jax import lax
from jax.experimental import pallas as pl
from jax.experimental.pallas import tpu as pltpu
from jax.experimental.pallas import tpu_sc as plsc

_D = 32
_L = 16
_CH = 1024
_NC = 2
_NS = 16
_NW = _NC * _NS


def _perm(v, idx):
    return lax.gather(
        v, idx[:, None],
        dimension_numbers=lax.GatherDimensionNumbers(
            offset_dims=(), collapsed_slice_dims=(0,), start_index_map=(0,)),
        slice_sizes=(1,),
        mode=lax.GatherScatterMode.PROMISE_IN_BOUNDS)


def _newton_rsqrt(x):
    i = lax.bitcast_convert_type(x, jnp.int32)
    i = jnp.int32(0x5F3759DF) - lax.shift_right_arithmetic(i, 1)
    y = lax.bitcast_convert_type(i, jnp.float32)
    for _ in range(3):
        y = y * (jnp.float32(1.5) - jnp.float32(0.5) * x * y * y)
    return y


def _normalize_chunk(rows_ref):
    lane = lax.iota(jnp.int32, _L)
    perms = [(lane ^ st).astype(jnp.int32) for st in (1, 2, 4, 8)]

    def group_body(i, carry):
        r0 = i * _L
        a = [rows_ref[r0 + r, pl.ds(0, _L)] for r in range(_L)]
        b = [rows_ref[r0 + r, pl.ds(_L, _L)] for r in range(_L)]
        S = jnp.zeros((_L,), jnp.float32)
        for r in range(_L):
            s = a[r] * a[r] + b[r] * b[r]
            for p in perms:
                s = s + _perm(s, p)
            S = jnp.where(lane == r, s, S)
        scale = _newton_rsqrt(S * jnp.float32(1.0 / _D) + jnp.float32(1e-8))
        for r in range(_L):
            sc = _perm(scale, jnp.full((_L,), r, jnp.int32))
            rows_ref[r0 + r, pl.ds(0, _L)] = a[r] * sc
            rows_ref[r0 + r, pl.ds(_L, _L)] = b[r] * sc
        return carry

    lax.fori_loop(0, _CH // _L, group_body, 0)


def _make_sc_kernel(n_rows):
    rows_per_worker = n_rows // _NW
    n_chunks = rows_per_worker // _CH
    assert rows_per_worker % _CH == 0

    mesh = plsc.VectorSubcoreMesh(core_axis_name="c", subcore_axis_name="s")

    @functools.partial(
        pl.kernel,
        out_type=jax.ShapeDtypeStruct((n_rows, _D), jnp.float32),
        mesh=mesh,
        scratch_types=[
            pltpu.VMEM((_CH,), jnp.int32),
            pltpu.VMEM((_CH,), jnp.int32),
            pltpu.VMEM((_CH, _D), jnp.float32),
            pltpu.VMEM((_CH, _D), jnp.float32),
            pltpu.SemaphoreType.DMA,
            pltpu.SemaphoreType.DMA,
            pltpu.SemaphoreType.DMA,
            pltpu.SemaphoreType.DMA,
        ],
        compiler_params=pltpu.CompilerParams(use_tc_tiling_on_sc=False),
    )
    def sc_kernel(idx_hbm, w_hbm, out_hbm,
                  idx0, idx1, rows0, rows1, sg0, sg1, so0, so1):
        wid = lax.axis_index("s") * _NC + lax.axis_index("c")
        base = wid * rows_per_worker

        idx_bufs = [idx0, idx1]
        row_bufs = [rows0, rows1]
        g_sems = [sg0, sg1]
        o_sems = [so0, so1]
        gather_h = [None, None]
        out_h = [None, None]

        def start_chunk(g):
            p = g % 2
            pltpu.sync_copy(idx_hbm.at[pl.ds(base + g * _CH, _CH)],
                            idx_bufs[p])
            gather_h[p] = pltpu.async_copy(
                w_hbm.at[idx_bufs[p]], row_bufs[p], g_sems[p])

        start_chunk(0)
        for g in range(n_chunks):
            p = g % 2
            if g + 1 < n_chunks:
                if out_h[1 - p] is not None:
                    out_h[1 - p].wait()
                    out_h[1 - p] = None
                start_chunk(g + 1)
            gather_h[p].wait()
            _normalize_chunk(row_bufs[p])
            out_h[p] = pltpu.async_copy(
                row_bufs[p], out_hbm.at[pl.ds(base + g * _CH, _CH)],
                o_sems[p])
        for p in (0, 1):
            if out_h[p] is not None:
                out_h[p].wait()

    return sc_kernel


def kernel(x, weight):
    orig_shape = x.shape
    idx = x.reshape(-1).astype(jnp.int32)
    out = _make_sc_kernel(idx.shape[0])(idx, weight)
    return out.reshape(*orig_shape, _D)


# speedup vs baseline: 2.1373x; 1.9346x over previous
import functools

import jax
import jax.numpy as jnp
from jax import lax
from jax.experimental import pallas as pl
from jax.experimental.pallas import tpu as pltpu
from jax.experimental.pallas import tpu_sc as plsc

_D = 32
_L = 16
_NC = 2
_NS = 16
_NW = _NC * _NS

_B = 16384
_T = 50
_TP = 56
_FAT = 128

_CB = 8
_CR = _CB * _T
_CF = _CB * _TP


def _perm(v, idx):
    return lax.gather(
        v, idx[:, None],
        dimension_numbers=lax.GatherDimensionNumbers(
            offset_dims=(), collapsed_slice_dims=(0,), start_index_map=(0,)),
        slice_sizes=(1,),
        mode=lax.GatherScatterMode.PROMISE_IN_BOUNDS)


def _newton_rsqrt(x):
    i = lax.bitcast_convert_type(x, jnp.int32)
    i = jnp.int32(0x5F3759DF) - lax.shift_right_arithmetic(i, 1)
    y = lax.bitcast_convert_type(i, jnp.float32)
    for _ in range(3):
        y = y * (jnp.float32(1.5) - jnp.float32(0.5) * x * y * y)
    return y


def _normalize_chunk_to_fat(rows_ref, fat_ref):
    lane = lax.iota(jnp.int32, _L)
    perms = [(lane ^ st).astype(jnp.int32) for st in (1, 2, 4, 8)]

    def group_body(i, carry):
        r0 = i * _L
        a = [rows_ref[r0 + r, pl.ds(0, _L)] for r in range(_L)]
        b = [rows_ref[r0 + r, pl.ds(_L, _L)] for r in range(_L)]
        S = jnp.zeros((_L,), jnp.float32)
        for r in range(_L):
            s = a[r] * a[r] + b[r] * b[r]
            for p in perms:
                s = s + _perm(s, p)
            S = jnp.where(lane == r, s, S)
        scale = _newton_rsqrt(S * jnp.float32(1.0 / _D) + jnp.float32(1e-8))
        for r in range(_L):
            sc = _perm(scale, jnp.full((_L,), r, jnp.int32))
            cr = r0 + r
            fr = cr + 6 * (cr // _T)
            fat_ref[fr, pl.ds(0, _L)] = a[r] * sc
            fat_ref[fr, pl.ds(_L, _L)] = b[r] * sc
        return carry

    lax.fori_loop(0, _CR // _L, group_body, 0)


def _make_sc_kernel():
    n_rows = _B * _T
    rows_per_worker = n_rows // _NW
    batches_per_worker = _B // _NW
    n_chunks = batches_per_worker // _CB
    fat_rows = _B * _TP

    mesh = plsc.VectorSubcoreMesh(core_axis_name="c", subcore_axis_name="s")

    @functools.partial(
        pl.kernel,
        out_type=jax.ShapeDtypeStruct((fat_rows, _FAT), jnp.float32),
        mesh=mesh,
        scratch_types=[
            pltpu.VMEM((_CR,), jnp.int32),
            pltpu.VMEM((_CR,), jnp.int32),
            pltpu.VMEM((_CR, _D), jnp.float32),
            pltpu.VMEM((_CR, _D), jnp.float32),
            pltpu.VMEM((_CF, _FAT), jnp.float32),
            pltpu.SemaphoreType.DMA,
            pltpu.SemaphoreType.DMA,
            pltpu.SemaphoreType.DMA,
        ],
        compiler_params=pltpu.CompilerParams(use_tc_tiling_on_sc=False),
    )
    def sc_kernel(idx_hbm, w_hbm, out_hbm,
                  idx0, idx1, rows0, rows1, fat, sg0, sg1, so):
        wid = lax.axis_index("s") * _NC + lax.axis_index("c")
        idx_base = wid * rows_per_worker
        fat_base = wid * batches_per_worker * _TP

        idx_bufs = [idx0, idx1]
        row_bufs = [rows0, rows1]
        g_sems = [sg0, sg1]

        def zero_body(i, carry):
            for c in range(_FAT // _L):
                fat[i, pl.ds(c * _L, _L)] = jnp.zeros((_L,), jnp.float32)
            return carry
        lax.fori_loop(0, _CF, zero_body, 0)

        def idx_load(g, p):
            pltpu.sync_copy(idx_hbm.at[pl.ds(idx_base + g * _CR, _CR)],
                            idx_bufs[p])

        def gather_start(g, p):
            pltpu.async_copy(w_hbm.at[idx_bufs[p]], row_bufs[p], g_sems[p])

        def gather_wait(p):
            pltpu.make_async_copy(w_hbm.at[idx_bufs[p]], row_bufs[p],
                                  g_sems[p]).wait()

        def out_start(g):
            pltpu.async_copy(
                fat, out_hbm.at[pl.ds(fat_base + g * _CF, _CF)], so)

        def out_wait(g):
            pltpu.make_async_copy(
                fat, out_hbm.at[pl.ds(fat_base + g * _CF, _CF)], so).wait()

        idx_load(0, 0)
        gather_start(0, 0)

        def pair_body(gp, carry):
            for sub in (0, 1):
                g = 2 * gp + sub

                @pl.when(g + 1 < n_chunks)
                def _():
                    idx_load(g + 1, 1 - sub)
                    gather_start(g + 1, 1 - sub)

                gather_wait(sub)

                @pl.when(g >= 1)
                def _():
                    out_wait(g - 1)

                _normalize_chunk_to_fat(row_bufs[sub], fat)
                out_start(g)
            return carry

        lax.fori_loop(0, n_chunks // 2, pair_body, 0)
        out_wait(n_chunks - 1)

    return sc_kernel


def kernel(x, weight):
    idx = x.reshape(-1).astype(jnp.int32)
    out_fat = _make_sc_kernel()(idx, weight)
    return out_fat.reshape(_B, _TP, _FAT)[:, :_T, :_D]


# speedup vs baseline: 2.1387x; 1.0007x over previous
import functools

import jax
import jax.numpy as jnp
from jax import lax
from jax.experimental import pallas as pl
from jax.experimental.pallas import tpu as pltpu
from jax.experimental.pallas import tpu_sc as plsc

_D = 32
_L = 16
_NC = 2
_NS = 16
_NW = _NC * _NS

_B = 16384
_T = 50
_TP = 56
_FAT = 128

_CB = 8
_CR = _CB * _T
_CF = _CB * _TP


def _perm(v, idx):
    return lax.gather(
        v, idx[:, None],
        dimension_numbers=lax.GatherDimensionNumbers(
            offset_dims=(), collapsed_slice_dims=(0,), start_index_map=(0,)),
        slice_sizes=(1,),
        mode=lax.GatherScatterMode.PROMISE_IN_BOUNDS)


def _newton_rsqrt(x):
    i = lax.bitcast_convert_type(x, jnp.int32)
    i = jnp.int32(0x5F3759DF) - lax.shift_right_arithmetic(i, 1)
    y = lax.bitcast_convert_type(i, jnp.float32)
    for _ in range(3):
        y = y * (jnp.float32(1.5) - jnp.float32(0.5) * x * y * y)
    return y


def _normalize_chunk_to_fat(rows_ref, fat_ref):
    lane = lax.iota(jnp.int32, _L)
    perms = [(lane ^ st).astype(jnp.int32) for st in (1, 2, 4, 8)]

    def group_body(i, carry):
        r0 = i * _L
        a = [rows_ref[r0 + r, pl.ds(0, _L)] for r in range(_L)]
        b = [rows_ref[r0 + r, pl.ds(_L, _L)] for r in range(_L)]
        S = jnp.zeros((_L,), jnp.float32)
        for r in range(_L):
            s = a[r] * a[r] + b[r] * b[r]
            for p in perms:
                s = s + _perm(s, p)
            S = jnp.where(lane == r, s, S)
        scale = _newton_rsqrt(S * jnp.float32(1.0 / _D) + jnp.float32(1e-8))
        for r in range(_L):
            sc = _perm(scale, jnp.full((_L,), r, jnp.int32))
            cr = r0 + r
            fr = cr + 6 * (cr // _T)
            fat_ref[fr, pl.ds(0, _L)] = a[r] * sc
            fat_ref[fr, pl.ds(_L, _L)] = b[r] * sc
        return carry

    lax.fori_loop(0, _CR // _L, group_body, 0)


def _make_sc_kernel():
    n_rows = _B * _T
    rows_per_worker = n_rows // _NW
    batches_per_worker = _B // _NW
    n_chunks = batches_per_worker // _CB
    fat_rows = _B * _TP

    mesh = plsc.VectorSubcoreMesh(core_axis_name="c", subcore_axis_name="s")

    @functools.partial(
        pl.kernel,
        out_type=jax.ShapeDtypeStruct((fat_rows, _FAT), jnp.float32),
        mesh=mesh,
        scratch_types=[
            pltpu.VMEM((_CR,), jnp.int32),
            pltpu.VMEM((_CR,), jnp.int32),
            pltpu.VMEM((_CR, _D), jnp.float32),
            pltpu.VMEM((_CR, _D), jnp.float32),
            pltpu.VMEM((_CF, _FAT), jnp.float32),
            pltpu.SemaphoreType.DMA,
            pltpu.SemaphoreType.DMA,
            pltpu.SemaphoreType.DMA,
        ],
        compiler_params=pltpu.CompilerParams(use_tc_tiling_on_sc=False),
    )
    def sc_kernel(idx_hbm, w_hbm, out_hbm,
                  idx0, idx1, rows0, rows1, fat, sg0, sg1, so):
        wid = lax.axis_index("s") * _NC + lax.axis_index("c")
        idx_base = wid * rows_per_worker
        fat_base = wid * batches_per_worker * _TP

        idx_bufs = [idx0, idx1]
        row_bufs = [rows0, rows1]
        g_sems = [sg0, sg1]

        def zero_body(i, carry):
            for c in range(_FAT // _L):
                fat[i, pl.ds(c * _L, _L)] = jnp.zeros((_L,), jnp.float32)
            return carry
        lax.fori_loop(0, _CF, zero_body, 0)

        def idx_load(g, p):
            pltpu.sync_copy(idx_hbm.at[pl.ds(idx_base + g * _CR, _CR)],
                            idx_bufs[p])

        def gather_start(g, p):
            pltpu.async_copy(w_hbm.at[idx_bufs[p]], row_bufs[p], g_sems[p])

        def gather_wait(p):
            pltpu.make_async_copy(w_hbm.at[idx_bufs[p]], row_bufs[p],
                                  g_sems[p]).wait()

        def out_start(g):
            pltpu.async_copy(
                fat, out_hbm.at[pl.ds(fat_base + g * _CF, _CF)], so)

        def out_wait(g):
            pltpu.make_async_copy(
                fat, out_hbm.at[pl.ds(fat_base + g * _CF, _CF)], so).wait()

        idx_load(0, 0)
        gather_start(0, 0)

        def pair_body(gp, carry):
            for sub in (0, 1):
                g = 2 * gp + sub

                @pl.when(g + 1 < n_chunks)
                def _():
                    idx_load(g + 1, 1 - sub)
                    gather_start(g + 1, 1 - sub)

                gather_wait(sub)

                @pl.when(g >= 1)
                def _():
                    out_wait(g - 1)

                _normalize_chunk_to_fat(row_bufs[sub], fat)
                out_start(g)
            return carry

        lax.fori_loop(0, n_chunks // 2, pair_body, 0)
        out_wait(n_chunks - 1)

    return sc_kernel


def kernel(x, weight):
    idx = x.reshape(-1).astype(jnp.int32)
    tbl = lax.optimization_barrier(weight.reshape(-1, 128))
    tbl = tbl.reshape(weight.shape[0], _D)
    out_fat = _make_sc_kernel()(idx, tbl)
    return out_fat.reshape(_B, _TP, _FAT)[:, :_T, :_D]
